# baseline (speedup 1.0000x reference)
import functools

import jax
import jax.numpy as jnp
import numpy as np
from jax import lax
from jax.experimental import pallas as pl
from jax.experimental.pallas import tpu as pltpu
from jax.experimental.pallas import tpu_sc as plsc

_NUM_POINTS = 512
_L = 16


def _np_linspace0(stop, num):
    div = num - 1
    step = np.arange(div, dtype=np.float32) / np.float32(div)
    body = np.float32(stop) * step
    return np.concatenate([body, np.array([stop], np.float32)])


def _int_tables(H, W, mh, mw):
    ys = _np_linspace0(float(mh - 1), H)
    xs = _np_linspace0(float(mw - 1), W)
    y0 = np.clip(np.floor(ys).astype(np.int32), 0, mh - 1)
    x0 = np.clip(np.floor(xs).astype(np.int32), 0, mw - 1)
    y1 = np.clip(y0 + 1, 0, mh - 1)
    x1 = np.clip(x0 + 1, 0, mw - 1)
    return y0, x0, y1, x1


def _sc_sample(featT, maskf, cls_flat, y0n, x0n, y1n, x1n, wy, wx,
               *, B, M, H, W, C, mh, mw):
    K = _NUM_POINTS // M
    P = H * W
    mask_sz = mh * mw
    n_pairs = B * M
    HALF = P // 2
    ROWS = H // 2
    CPR = W // _L
    KH = K // 2
    NSUB = 16

    nrow0 = int(y1n[ROWS - 1]) + 1
    lo1 = int(y0n[ROWS])
    nrows = max(nrow0, mh - lo1)
    lo_by_half = (0, min(lo1, mh - nrows))

    mesh = plsc.VectorSubcoreMesh(core_axis_name="c", subcore_axis_name="s",
                                  num_cores=2, num_subcores=NSUB)

    @functools.partial(
        pl.kernel,
        out_type=(
            jax.ShapeDtypeStruct((B * _NUM_POINTS, C), jnp.float32),
            jax.ShapeDtypeStruct((3 * B * _NUM_POINTS,), jnp.int32),
        ),
        mesh=mesh,
        compiler_params=pltpu.CompilerParams(needs_layout_passes=False),
        scratch_types=[
            pltpu.VMEM((nrows, mw), jnp.float32),
            pltpu.VMEM((H,), jnp.int32),
            pltpu.VMEM((H,), jnp.int32),
            pltpu.VMEM((W,), jnp.int32),
            pltpu.VMEM((W,), jnp.int32),
            pltpu.VMEM((H,), jnp.float32),
            pltpu.VMEM((W,), jnp.float32),
            pltpu.VMEM((K,), jnp.int32),
            pltpu.VMEM((K,), jnp.int32),
            pltpu.VMEM((_L,), jnp.int32),
            pltpu.VMEM((K,), jnp.int32),
            pltpu.VMEM((K,), jnp.int32),
            pltpu.VMEM((K,), jnp.int32),
            pltpu.VMEM((_L,), jnp.int32),
            pltpu.VMEM((_L,), jnp.int32),
            pltpu.VMEM((KH,), jnp.int32),
            pltpu.VMEM((KH,), jnp.int32),
            pltpu.VMEM((KH, C), jnp.float32),
            pltpu.VMEM((1, C), jnp.float32),
            pltpu.VMEM((KH,), jnp.int32),
            pltpu.VMEM((KH,), jnp.int32),
            pltpu.VMEM((KH,), jnp.int32),
            pltpu.VMEM_SHARED((NSUB * K,), jnp.int32),
            pltpu.VMEM_SHARED((NSUB * K,), jnp.int32),
            pltpu.VMEM_SHARED((NSUB * _L,), jnp.int32),
            pltpu.SemaphoreType.DMA,
        ],
    )
    def body(featT_h, maskf_h, cls_h, y0_h, y1_h, x0_h, x1_h, wy_h, wx_h,
             points_h, indices_h, mask_v, y0_v, y1_v, x0_v, x1_v,
             wy_v, wx_v, tloc_v, floc_v, cnt_v, t0_v, t1_v, f0_v, c0_v,
             c1_v, oidx_v, gidx_v, rows_v, cls_v, hh_v, ww_v, bb_v,
             sh_t, sh_f, sh_c, sem):
        s = lax.axis_index("s")
        core = lax.axis_index("c")
        pair = core * (n_pairs // 2) + (s // 2)
        half = s % 2
        b = pair // M
        j = pair % M

        rowlo = jnp.where(half == 0, lo_by_half[0], lo_by_half[1])

        ins = [
            pltpu.async_copy(
                maskf_h.at[b, j, pl.ds(rowlo, nrows), :], mask_v, sem),
            pltpu.async_copy(y0_h, y0_v, sem),
            pltpu.async_copy(y1_h, y1_v, sem),
            pltpu.async_copy(x0_h, x0_v, sem),
            pltpu.async_copy(x1_h, x1_v, sem),
            pltpu.async_copy(wy_h, wy_v, sem),
            pltpu.async_copy(wx_h, wx_v, sem),
            pltpu.async_copy(cls_h.at[pl.ds(0, 1), :], cls_v, sem),
        ]
        for h in ins:
            h.wait()

        iota = lax.broadcasted_iota(jnp.int32, (_L,), 0)
        zero = jnp.zeros((_L,), jnp.int32)
        x0c = [x0_v[pl.ds(q * _L, _L)] for q in range(CPR)]
        x1c = [x1_v[pl.ds(q * _L, _L)] for q in range(CPR)]
        wxc = [wx_v[pl.ds(q * _L, _L)] for q in range(CPR)]
        axc = [1.0 - w for w in wxc]

        def row_body(r, ct):
            y = half * ROWS + r
            zy = zero + y
            y0s = plsc.load_gather(y0_v, [zy])
            y1s = plsc.load_gather(y1_v, [zy])
            wys = plsc.load_gather(wy_v, [zy])
            y0r = y0s - rowlo
            y1r = y1s - rowlo
            ay = 1.0 - wys
            for q in range(CPR):
                v00 = plsc.load_gather(mask_v, [y0r, x0c[q]])
                v01 = plsc.load_gather(mask_v, [y0r, x1c[q]])
                v10 = plsc.load_gather(mask_v, [y1r, x0c[q]])
                v11 = plsc.load_gather(mask_v, [y1r, x1c[q]])
                t1 = (v00 * ay) * axc[q]
                t2 = (v01 * ay) * wxc[q]
                t3 = (v10 * wys) * axc[q]
                t4 = (v11 * wys) * wxc[q]
                val = ((t1 + t2) + t3) + t4
                m = val > 0.5
                mi = m.astype(jnp.int32)
                lpos = (r * W + q * _L) + iota
                rt = jnp.cumsum(mi) + ct
                rf = (lpos + 1) - rt
                st = jnp.clip(rt - 1, 0, K - 1)
                sf = jnp.clip(rf - 1, 0, K - 1)
                gpos = lpos + half * HALF
                plsc.store_scatter(tloc_v, [st], gpos, mask=m & (rt <= K))
                plsc.store_scatter(floc_v, [sf], gpos,
                                   mask=(~m) & (rf <= K))
                ct = ct + plsc.all_reduce_population_count(m)
            return ct

        Tv = lax.fori_loop(0, ROWS, row_body,
                           jnp.zeros((_L,), jnp.int32), unroll=4)

        cnt_v[...] = Tv
        pltpu.sync_copy(tloc_v, sh_t.at[pl.ds(s * K, K)])
        pltpu.sync_copy(floc_v, sh_f.at[pl.ds(s * K, K)])
        pltpu.sync_copy(cnt_v, sh_c.at[pl.ds(s * _L, _L)])
        plsc.subcore_barrier()
        lo = (s // 2) * 2
        hi = lo + 1
        reads = [
            pltpu.async_copy(sh_t.at[pl.ds(lo * K, K)], t0_v, sem),
            pltpu.async_copy(sh_t.at[pl.ds(hi * K, K)], t1_v, sem),
            pltpu.async_copy(sh_f.at[pl.ds(lo * K, K)], f0_v, sem),
            pltpu.async_copy(sh_c.at[pl.ds(lo * _L, _L)], c0_v, sem),
            pltpu.async_copy(sh_c.at[pl.ds(hi * _L, _L)], c1_v, sem),
        ]
        for h in reads:
            h.wait()

        T0 = c0_v[...]
        T1 = c1_v[...]
        b1 = T0 + T1
        log2w = W.bit_length() - 1
        for q in range(KH // _L):
            sv = (half * KH + q * _L) + iota
            in0 = sv < T0
            in1 = (~in0) & (sv < b1)
            in2 = sv >= b1
            g0 = plsc.load_gather(t0_v, [jnp.clip(sv, 0, K - 1)], mask=in0)
            g1 = plsc.load_gather(t1_v, [jnp.clip(sv - T0, 0, K - 1)],
                                  mask=in1)
            g2 = plsc.load_gather(f0_v, [jnp.clip(sv - b1, 0, K - 1)],
                                  mask=in2)
            iv = jnp.where(in0, g0, jnp.where(in1, g1, g2))
            sl = pl.ds(q * _L, _L)
            oidx_v[sl] = iv
            gidx_v[sl] = iv + b * P
            hh_v[sl] = lax.shift_right_logical(iv, log2w)
            ww_v[sl] = iv & (W - 1)
            bb_v[sl] = zero + b

        pltpu.async_copy(featT_h.at[gidx_v], rows_v, sem).wait()

        cls_chunks = [cls_v[0, pl.ds(t * _L, _L)] for t in range(C // _L)]

        def addrow(r, carry):
            for t in range(C // _L):
                rows_v[r, pl.ds(t * _L, _L)] += cls_chunks[t]
            return carry

        lax.fori_loop(0, KH, addrow, jnp.int32(0))

        col = pair * K + half * KH
        npts = B * _NUM_POINTS
        outs = [
            pltpu.async_copy(rows_v, points_h.at[pl.ds(col, KH)], sem),
            pltpu.async_copy(bb_v, indices_h.at[pl.ds(col, KH)], sem),
            pltpu.async_copy(hh_v, indices_h.at[pl.ds(npts + col, KH)],
                             sem),
            pltpu.async_copy(ww_v,
                             indices_h.at[pl.ds(2 * npts + col, KH)], sem),
        ]
        for h in outs:
            h.wait()

    return body(featT, maskf, cls_flat, y0n, y1n, x0n, x1n, wy, wx)


def kernel(feat, ior_masks, cls_table):
    B, C, H, W = feat.shape
    M = ior_masks.shape[1]
    mh, mw = ior_masks.shape[2], ior_masks.shape[3]

    featT = feat.transpose(0, 2, 3, 1).reshape(B * H * W, C)

    y0n, x0n, y1n, x1n = _int_tables(H, W, mh, mw)

    ys = jnp.linspace(0.0, float(mh - 1), H)
    xs = jnp.linspace(0.0, float(mw - 1), W)
    wy = ys - jnp.asarray(y0n).astype(jnp.float32)
    wx = xs - jnp.asarray(x0n).astype(jnp.float32)

    points_flat, indices_flat = _sc_sample(
        featT, ior_masks, cls_table, y0n, x0n, y1n, x1n, wy, wx,
        B=B, M=M, H=H, W=W, C=C, mh=mh, mw=mw)
    return (points_flat.reshape(B, _NUM_POINTS, C),
            indices_flat.reshape(3, B * _NUM_POINTS))


# speedup vs baseline: 1.0151x; 1.0151x over previous
import functools

import jax
import jax.numpy as jnp
import numpy as np
from jax import lax
from jax.experimental import pallas as pl
from jax.experimental.pallas import tpu as pltpu
from jax.experimental.pallas import tpu_sc as plsc

_NUM_POINTS = 512
_L = 16


def _np_linspace0(stop, num):
    div = num - 1
    step = np.arange(div, dtype=np.float32) / np.float32(div)
    body = np.float32(stop) * step
    return np.concatenate([body, np.array([stop], np.float32)])


def _int_tables(H, W, mh, mw):
    ys = _np_linspace0(float(mh - 1), H)
    xs = _np_linspace0(float(mw - 1), W)
    y0 = np.clip(np.floor(ys).astype(np.int32), 0, mh - 1)
    x0 = np.clip(np.floor(xs).astype(np.int32), 0, mw - 1)
    y1 = np.clip(y0 + 1, 0, mh - 1)
    x1 = np.clip(x0 + 1, 0, mw - 1)
    return y0, x0, y1, x1


def _jax_weights(H, W, mh, mw, y0n, x0n):
    def f():
        ys = jnp.linspace(0.0, float(mh - 1), H)
        xs = jnp.linspace(0.0, float(mw - 1), W)
        wy = ys - jnp.asarray(y0n).astype(jnp.float32)
        wx = xs - jnp.asarray(x0n).astype(jnp.float32)
        return wy, wx
    return f


try:
    _W_6464 = tuple(
        np.asarray(a)
        for a in jax.jit(_jax_weights(64, 64, 128, 128,
                                      *_int_tables(64, 64, 128, 128)[:2]))())
except Exception:
    _W_6464 = None


def _sc_sample(featT, maskf, cls_flat, y0n, x0n, y1n, x1n, wy, wx,
               *, B, M, H, W, C, mh, mw):
    K = _NUM_POINTS // M
    P = H * W
    mask_sz = mh * mw
    n_pairs = B * M
    HALF = P // 2
    ROWS = H // 2
    CPR = W // _L
    KH = K // 2
    NSUB = 16

    nrow0 = int(y1n[ROWS - 1]) + 1
    lo1 = int(y0n[ROWS])
    nrows = max(nrow0, mh - lo1)
    lo_by_half = (0, min(lo1, mh - nrows))

    mesh = plsc.VectorSubcoreMesh(core_axis_name="c", subcore_axis_name="s",
                                  num_cores=2, num_subcores=NSUB)

    @functools.partial(
        pl.kernel,
        out_type=(
            jax.ShapeDtypeStruct((B * _NUM_POINTS, C), jnp.float32),
            jax.ShapeDtypeStruct((3 * B * _NUM_POINTS,), jnp.int32),
        ),
        mesh=mesh,
        compiler_params=pltpu.CompilerParams(needs_layout_passes=False),
        scratch_types=[
            pltpu.VMEM((nrows, mw), jnp.float32),
            pltpu.VMEM((H,), jnp.int32),
            pltpu.VMEM((H,), jnp.int32),
            pltpu.VMEM((W,), jnp.int32),
            pltpu.VMEM((W,), jnp.int32),
            pltpu.VMEM((H,), jnp.float32),
            pltpu.VMEM((W,), jnp.float32),
            pltpu.VMEM((K,), jnp.int32),
            pltpu.VMEM((K,), jnp.int32),
            pltpu.VMEM((_L,), jnp.int32),
            pltpu.VMEM((K,), jnp.int32),
            pltpu.VMEM((K,), jnp.int32),
            pltpu.VMEM((K,), jnp.int32),
            pltpu.VMEM((_L,), jnp.int32),
            pltpu.VMEM((_L,), jnp.int32),
            pltpu.VMEM((KH,), jnp.int32),
            pltpu.VMEM((KH,), jnp.int32),
            pltpu.VMEM((KH, C), jnp.float32),
            pltpu.VMEM((1, C), jnp.float32),
            pltpu.VMEM((KH,), jnp.int32),
            pltpu.VMEM((KH,), jnp.int32),
            pltpu.VMEM((KH,), jnp.int32),
            pltpu.VMEM_SHARED((NSUB * K,), jnp.int32),
            pltpu.VMEM_SHARED((NSUB * K,), jnp.int32),
            pltpu.VMEM_SHARED((NSUB * _L,), jnp.int32),
            pltpu.SemaphoreType.DMA,
        ],
    )
    def body(featT_h, maskf_h, cls_h, y0_h, y1_h, x0_h, x1_h, wy_h, wx_h,
             points_h, indices_h, mask_v, y0_v, y1_v, x0_v, x1_v,
             wy_v, wx_v, tloc_v, floc_v, cnt_v, t0_v, t1_v, f0_v, c0_v,
             c1_v, oidx_v, gidx_v, rows_v, cls_v, hh_v, ww_v, bb_v,
             sh_t, sh_f, sh_c, sem):
        s = lax.axis_index("s")
        core = lax.axis_index("c")
        pair = core * (n_pairs // 2) + (s // 2)
        half = s % 2
        b = pair // M
        j = pair % M

        rowlo = jnp.where(half == 0, lo_by_half[0], lo_by_half[1])

        ins = [
            pltpu.async_copy(
                maskf_h.at[b, j, pl.ds(rowlo, nrows), :], mask_v, sem),
            pltpu.async_copy(y0_h, y0_v, sem),
            pltpu.async_copy(y1_h, y1_v, sem),
            pltpu.async_copy(x0_h, x0_v, sem),
            pltpu.async_copy(x1_h, x1_v, sem),
            pltpu.async_copy(wy_h, wy_v, sem),
            pltpu.async_copy(wx_h, wx_v, sem),
            pltpu.async_copy(cls_h.at[pl.ds(0, 1), :], cls_v, sem),
        ]
        for h in ins:
            h.wait()

        iota = lax.broadcasted_iota(jnp.int32, (_L,), 0)
        zero = jnp.zeros((_L,), jnp.int32)
        x0c = [x0_v[pl.ds(q * _L, _L)] for q in range(CPR)]
        x1c = [x1_v[pl.ds(q * _L, _L)] for q in range(CPR)]
        wxc = [wx_v[pl.ds(q * _L, _L)] for q in range(CPR)]
        axc = [1.0 - w for w in wxc]

        def row_body(r, ct):
            y = half * ROWS + r
            zy = zero + y
            y0s = plsc.load_gather(y0_v, [zy])
            y1s = plsc.load_gather(y1_v, [zy])
            wys = plsc.load_gather(wy_v, [zy])
            y0r = y0s - rowlo
            y1r = y1s - rowlo
            ay = 1.0 - wys
            for q in range(CPR):
                v00 = plsc.load_gather(mask_v, [y0r, x0c[q]])
                v01 = plsc.load_gather(mask_v, [y0r, x1c[q]])
                v10 = plsc.load_gather(mask_v, [y1r, x0c[q]])
                v11 = plsc.load_gather(mask_v, [y1r, x1c[q]])
                t1 = (v00 * ay) * axc[q]
                t2 = (v01 * ay) * wxc[q]
                t3 = (v10 * wys) * axc[q]
                t4 = (v11 * wys) * wxc[q]
                val = ((t1 + t2) + t3) + t4
                m = val > 0.5
                mi = m.astype(jnp.int32)
                lpos = (r * W + q * _L) + iota
                rt = jnp.cumsum(mi) + ct
                rf = (lpos + 1) - rt
                st = jnp.clip(rt - 1, 0, K - 1)
                sf = jnp.clip(rf - 1, 0, K - 1)
                gpos = lpos + half * HALF
                plsc.store_scatter(tloc_v, [st], gpos, mask=m & (rt <= K))
                plsc.store_scatter(floc_v, [sf], gpos,
                                   mask=(~m) & (rf <= K))
                ct = ct + plsc.all_reduce_population_count(m)
            return ct

        Tv = lax.fori_loop(0, ROWS, row_body,
                           jnp.zeros((_L,), jnp.int32), unroll=2)

        cnt_v[...] = Tv
        pltpu.sync_copy(tloc_v, sh_t.at[pl.ds(s * K, K)])
        pltpu.sync_copy(floc_v, sh_f.at[pl.ds(s * K, K)])
        pltpu.sync_copy(cnt_v, sh_c.at[pl.ds(s * _L, _L)])
        plsc.subcore_barrier()
        lo = (s // 2) * 2
        hi = lo + 1
        reads = [
            pltpu.async_copy(sh_t.at[pl.ds(lo * K, K)], t0_v, sem),
            pltpu.async_copy(sh_t.at[pl.ds(hi * K, K)], t1_v, sem),
            pltpu.async_copy(sh_f.at[pl.ds(lo * K, K)], f0_v, sem),
            pltpu.async_copy(sh_c.at[pl.ds(lo * _L, _L)], c0_v, sem),
            pltpu.async_copy(sh_c.at[pl.ds(hi * _L, _L)], c1_v, sem),
        ]
        for h in reads:
            h.wait()

        T0 = c0_v[...]
        T1 = c1_v[...]
        b1 = T0 + T1
        log2w = W.bit_length() - 1
        for q in range(KH // _L):
            sv = (half * KH + q * _L) + iota
            in0 = sv < T0
            in1 = (~in0) & (sv < b1)
            in2 = sv >= b1
            g0 = plsc.load_gather(t0_v, [jnp.clip(sv, 0, K - 1)], mask=in0)
            g1 = plsc.load_gather(t1_v, [jnp.clip(sv - T0, 0, K - 1)],
                                  mask=in1)
            g2 = plsc.load_gather(f0_v, [jnp.clip(sv - b1, 0, K - 1)],
                                  mask=in2)
            iv = jnp.where(in0, g0, jnp.where(in1, g1, g2))
            sl = pl.ds(q * _L, _L)
            oidx_v[sl] = iv
            gidx_v[sl] = iv + b * P
            hh_v[sl] = lax.shift_right_logical(iv, log2w)
            ww_v[sl] = iv & (W - 1)
            bb_v[sl] = zero + b

        pltpu.async_copy(featT_h.at[gidx_v], rows_v, sem).wait()

        cls_chunks = [cls_v[0, pl.ds(t * _L, _L)] for t in range(C // _L)]

        def addrow(r, carry):
            for t in range(C // _L):
                rows_v[r, pl.ds(t * _L, _L)] += cls_chunks[t]
            return carry

        lax.fori_loop(0, KH, addrow, jnp.int32(0))

        col = pair * K + half * KH
        npts = B * _NUM_POINTS
        outs = [
            pltpu.async_copy(rows_v, points_h.at[pl.ds(col, KH)], sem),
            pltpu.async_copy(bb_v, indices_h.at[pl.ds(col, KH)], sem),
            pltpu.async_copy(hh_v, indices_h.at[pl.ds(npts + col, KH)],
                             sem),
            pltpu.async_copy(ww_v,
                             indices_h.at[pl.ds(2 * npts + col, KH)], sem),
        ]
        for h in outs:
            h.wait()

    return body(featT, maskf, cls_flat, y0n, y1n, x0n, x1n, wy, wx)


def kernel(feat, ior_masks, cls_table):
    B, C, H, W = feat.shape
    M = ior_masks.shape[1]
    mh, mw = ior_masks.shape[2], ior_masks.shape[3]

    featT = feat.transpose(0, 2, 3, 1).reshape(B * H * W, C)

    y0n, x0n, y1n, x1n = _int_tables(H, W, mh, mw)

    if (H, W, mh, mw) == (64, 64, 128, 128) and _W_6464 is not None:
        wy, wx = _W_6464
    else:
        wy, wx = _jax_weights(H, W, mh, mw, y0n, x0n)()

    points_flat, indices_flat = _sc_sample(
        featT, ior_masks, cls_table, y0n, x0n, y1n, x1n, wy, wx,
        B=B, M=M, H=H, W=W, C=C, mh=mh, mw=mw)
    return (points_flat.reshape(B, _NUM_POINTS, C),
            indices_flat.reshape(3, B * _NUM_POINTS))


# speedup vs baseline: 1.0236x; 1.0083x over previous
import functools

import jax
import jax.numpy as jnp
import numpy as np
from jax import lax
from jax.experimental import pallas as pl
from jax.experimental.pallas import tpu as pltpu
from jax.experimental.pallas import tpu_sc as plsc

_NUM_POINTS = 512
_L = 16


def _np_linspace0(stop, num):
    div = num - 1
    step = np.arange(div, dtype=np.float32) / np.float32(div)
    body = np.float32(stop) * step
    return np.concatenate([body, np.array([stop], np.float32)])


def _int_tables(H, W, mh, mw):
    ys = _np_linspace0(float(mh - 1), H)
    xs = _np_linspace0(float(mw - 1), W)
    y0 = np.clip(np.floor(ys).astype(np.int32), 0, mh - 1)
    x0 = np.clip(np.floor(xs).astype(np.int32), 0, mw - 1)
    y1 = np.clip(y0 + 1, 0, mh - 1)
    x1 = np.clip(x0 + 1, 0, mw - 1)
    return y0, x0, y1, x1


def _jax_weights(H, W, mh, mw, y0n, x0n):
    def f():
        ys = jnp.linspace(0.0, float(mh - 1), H)
        xs = jnp.linspace(0.0, float(mw - 1), W)
        wy = ys - jnp.asarray(y0n).astype(jnp.float32)
        wx = xs - jnp.asarray(x0n).astype(jnp.float32)
        return wy, wx
    return f


try:
    _W_6464 = tuple(
        np.asarray(a)
        for a in jax.jit(_jax_weights(64, 64, 128, 128,
                                      *_int_tables(64, 64, 128, 128)[:2]))())
except Exception:
    _W_6464 = None


def _sc_sample(featT, maskf, cls_flat, y0n, x0n, y1n, x1n, wy, wx,
               *, B, M, H, W, C, mh, mw):
    K = _NUM_POINTS // M
    P = H * W
    mask_sz = mh * mw
    n_pairs = B * M
    HALF = P // 2
    ROWS = H // 2
    CPR = W // _L
    KH = K // 2
    NSUB = 16

    nrow0 = int(y1n[ROWS - 1]) + 1
    lo1 = int(y0n[ROWS])
    nrows = max(nrow0, mh - lo1)
    lo_by_half = (0, min(lo1, mh - nrows))

    mesh = plsc.VectorSubcoreMesh(core_axis_name="c", subcore_axis_name="s",
                                  num_cores=2, num_subcores=NSUB)

    @functools.partial(
        pl.kernel,
        out_type=(
            jax.ShapeDtypeStruct((B * _NUM_POINTS, C), jnp.float32),
            jax.ShapeDtypeStruct((3 * B * _NUM_POINTS,), jnp.int32),
        ),
        mesh=mesh,
        compiler_params=pltpu.CompilerParams(needs_layout_passes=False),
        scratch_types=[
            pltpu.VMEM((nrows, mw), jnp.float32),
            pltpu.VMEM((H,), jnp.int32),
            pltpu.VMEM((H,), jnp.int32),
            pltpu.VMEM((W,), jnp.int32),
            pltpu.VMEM((W,), jnp.int32),
            pltpu.VMEM((H,), jnp.float32),
            pltpu.VMEM((W,), jnp.float32),
            pltpu.VMEM((K,), jnp.int32),
            pltpu.VMEM((K,), jnp.int32),
            pltpu.VMEM((_L,), jnp.int32),
            pltpu.VMEM((K,), jnp.int32),
            pltpu.VMEM((K,), jnp.int32),
            pltpu.VMEM((K,), jnp.int32),
            pltpu.VMEM((_L,), jnp.int32),
            pltpu.VMEM((_L,), jnp.int32),
            pltpu.VMEM((KH,), jnp.int32),
            pltpu.VMEM((KH,), jnp.int32),
            pltpu.VMEM((KH, C), jnp.float32),
            pltpu.VMEM((1, C), jnp.float32),
            pltpu.VMEM((KH,), jnp.int32),
            pltpu.VMEM((KH,), jnp.int32),
            pltpu.VMEM((KH,), jnp.int32),
            pltpu.VMEM_SHARED((NSUB * K,), jnp.int32),
            pltpu.VMEM_SHARED((NSUB * K,), jnp.int32),
            pltpu.VMEM_SHARED((NSUB * _L,), jnp.int32),
            pltpu.SemaphoreType.DMA,
        ],
    )
    def body(featT_h, maskf_h, cls_h, y0_h, y1_h, x0_h, x1_h, wy_h, wx_h,
             points_h, indices_h, mask_v, y0_v, y1_v, x0_v, x1_v,
             wy_v, wx_v, tloc_v, floc_v, cnt_v, t0_v, t1_v, f0_v, c0_v,
             c1_v, oidx_v, gidx_v, rows_v, cls_v, hh_v, ww_v, bb_v,
             sh_t, sh_f, sh_c, sem):
        s = lax.axis_index("s")
        core = lax.axis_index("c")
        pair = core * (n_pairs // 2) + (s // 2)
        half = s % 2
        b = pair // M
        j = pair % M

        rowlo = jnp.where(half == 0, lo_by_half[0], lo_by_half[1])

        ins = [
            pltpu.async_copy(
                maskf_h.at[b, j, pl.ds(rowlo, nrows), :], mask_v, sem),
            pltpu.async_copy(y0_h, y0_v, sem),
            pltpu.async_copy(y1_h, y1_v, sem),
            pltpu.async_copy(x0_h, x0_v, sem),
            pltpu.async_copy(x1_h, x1_v, sem),
            pltpu.async_copy(wy_h, wy_v, sem),
            pltpu.async_copy(wx_h, wx_v, sem),
            pltpu.async_copy(cls_h.at[pl.ds(0, 1), :], cls_v, sem),
        ]
        for h in ins:
            h.wait()

        iota = lax.broadcasted_iota(jnp.int32, (_L,), 0)
        zero = jnp.zeros((_L,), jnp.int32)
        x0c = [x0_v[pl.ds(q * _L, _L)] for q in range(CPR)]
        x1c = [x1_v[pl.ds(q * _L, _L)] for q in range(CPR)]
        wxc = [wx_v[pl.ds(q * _L, _L)] for q in range(CPR)]
        axc = [1.0 - w for w in wxc]

        def row_body(r, ct):
            y = half * ROWS + r
            zy = zero + y
            y0s = plsc.load_gather(y0_v, [zy])
            y1s = plsc.load_gather(y1_v, [zy])
            wys = plsc.load_gather(wy_v, [zy])
            y0r = y0s - rowlo
            y1r = y1s - rowlo
            ay = 1.0 - wys
            for q in range(CPR):
                v00 = plsc.load_gather(mask_v, [y0r, x0c[q]])
                v01 = plsc.load_gather(mask_v, [y0r, x1c[q]])
                v10 = plsc.load_gather(mask_v, [y1r, x0c[q]])
                v11 = plsc.load_gather(mask_v, [y1r, x1c[q]])
                t1 = (v00 * ay) * axc[q]
                t2 = (v01 * ay) * wxc[q]
                t3 = (v10 * wys) * axc[q]
                t4 = (v11 * wys) * wxc[q]
                val = ((t1 + t2) + t3) + t4
                m = val > 0.5
                mi = m.astype(jnp.int32)
                lpos = (r * W + q * _L) + iota
                rt = jnp.cumsum(mi) + ct
                rf = (lpos + 1) - rt
                st = jnp.clip(rt - 1, 0, K - 1)
                sf = jnp.clip(rf - 1, 0, K - 1)
                gpos = lpos + half * HALF
                plsc.store_scatter(tloc_v, [st], gpos, mask=m & (rt <= K))
                plsc.store_scatter(floc_v, [sf], gpos,
                                   mask=(~m) & (rf <= K))
                ct = ct + plsc.all_reduce_population_count(m)
            return ct

        Tv = plsc.parallel_loop(
            0, ROWS, 1, unroll=2,
            carry=jnp.zeros((_L,), jnp.int32))(row_body)

        cnt_v[...] = Tv
        pltpu.sync_copy(tloc_v, sh_t.at[pl.ds(s * K, K)])
        pltpu.sync_copy(floc_v, sh_f.at[pl.ds(s * K, K)])
        pltpu.sync_copy(cnt_v, sh_c.at[pl.ds(s * _L, _L)])
        plsc.subcore_barrier()
        lo = (s // 2) * 2
        hi = lo + 1
        reads = [
            pltpu.async_copy(sh_t.at[pl.ds(lo * K, K)], t0_v, sem),
            pltpu.async_copy(sh_t.at[pl.ds(hi * K, K)], t1_v, sem),
            pltpu.async_copy(sh_f.at[pl.ds(lo * K, K)], f0_v, sem),
            pltpu.async_copy(sh_c.at[pl.ds(lo * _L, _L)], c0_v, sem),
            pltpu.async_copy(sh_c.at[pl.ds(hi * _L, _L)], c1_v, sem),
        ]
        for h in reads:
            h.wait()

        T0 = c0_v[...]
        T1 = c1_v[...]
        b1 = T0 + T1
        log2w = W.bit_length() - 1
        for q in range(KH // _L):
            sv = (half * KH + q * _L) + iota
            in0 = sv < T0
            in1 = (~in0) & (sv < b1)
            in2 = sv >= b1
            g0 = plsc.load_gather(t0_v, [jnp.clip(sv, 0, K - 1)], mask=in0)
            g1 = plsc.load_gather(t1_v, [jnp.clip(sv - T0, 0, K - 1)],
                                  mask=in1)
            g2 = plsc.load_gather(f0_v, [jnp.clip(sv - b1, 0, K - 1)],
                                  mask=in2)
            iv = jnp.where(in0, g0, jnp.where(in1, g1, g2))
            sl = pl.ds(q * _L, _L)
            oidx_v[sl] = iv
            gidx_v[sl] = iv + b * P
            hh_v[sl] = lax.shift_right_logical(iv, log2w)
            ww_v[sl] = iv & (W - 1)
            bb_v[sl] = zero + b

        pltpu.async_copy(featT_h.at[gidx_v], rows_v, sem).wait()

        cls_chunks = [cls_v[0, pl.ds(t * _L, _L)] for t in range(C // _L)]

        def addrow(r, carry):
            for t in range(C // _L):
                rows_v[r, pl.ds(t * _L, _L)] += cls_chunks[t]
            return carry

        lax.fori_loop(0, KH, addrow, jnp.int32(0))

        col = pair * K + half * KH
        npts = B * _NUM_POINTS
        outs = [
            pltpu.async_copy(rows_v, points_h.at[pl.ds(col, KH)], sem),
            pltpu.async_copy(bb_v, indices_h.at[pl.ds(col, KH)], sem),
            pltpu.async_copy(hh_v, indices_h.at[pl.ds(npts + col, KH)],
                             sem),
            pltpu.async_copy(ww_v,
                             indices_h.at[pl.ds(2 * npts + col, KH)], sem),
        ]
        for h in outs:
            h.wait()

    return body(featT, maskf, cls_flat, y0n, y1n, x0n, x1n, wy, wx)


def kernel(feat, ior_masks, cls_table):
    B, C, H, W = feat.shape
    M = ior_masks.shape[1]
    mh, mw = ior_masks.shape[2], ior_masks.shape[3]

    featT = feat.transpose(0, 2, 3, 1).reshape(B * H * W, C)

    y0n, x0n, y1n, x1n = _int_tables(H, W, mh, mw)

    if (H, W, mh, mw) == (64, 64, 128, 128) and _W_6464 is not None:
        wy, wx = _W_6464
    else:
        wy, wx = _jax_weights(H, W, mh, mw, y0n, x0n)()

    points_flat, indices_flat = _sc_sample(
        featT, ior_masks, cls_table, y0n, x0n, y1n, x1n, wy, wx,
        B=B, M=M, H=H, W=W, C=C, mh=mh, mw=mw)
    return (points_flat.reshape(B, _NUM_POINTS, C),
            indices_flat.reshape(3, B * _NUM_POINTS))


# speedup vs baseline: 1.0242x; 1.0007x over previous
import functools

import jax
import jax.numpy as jnp
import numpy as np
from jax import lax
from jax.experimental import pallas as pl
from jax.experimental.pallas import tpu as pltpu
from jax.experimental.pallas import tpu_sc as plsc

_NUM_POINTS = 512
_L = 16


def _np_linspace0(stop, num):
    div = num - 1
    step = np.arange(div, dtype=np.float32) / np.float32(div)
    body = np.float32(stop) * step
    return np.concatenate([body, np.array([stop], np.float32)])


def _int_tables(H, W, mh, mw):
    ys = _np_linspace0(float(mh - 1), H)
    xs = _np_linspace0(float(mw - 1), W)
    y0 = np.clip(np.floor(ys).astype(np.int32), 0, mh - 1)
    x0 = np.clip(np.floor(xs).astype(np.int32), 0, mw - 1)
    y1 = np.clip(y0 + 1, 0, mh - 1)
    x1 = np.clip(x0 + 1, 0, mw - 1)
    return y0, x0, y1, x1


def _jax_weights(H, W, mh, mw, y0n, x0n):
    def f():
        ys = jnp.linspace(0.0, float(mh - 1), H)
        xs = jnp.linspace(0.0, float(mw - 1), W)
        wy = ys - jnp.asarray(y0n).astype(jnp.float32)
        wx = xs - jnp.asarray(x0n).astype(jnp.float32)
        return wy, wx
    return f


try:
    _W_6464 = tuple(
        np.asarray(a)
        for a in jax.jit(_jax_weights(64, 64, 128, 128,
                                      *_int_tables(64, 64, 128, 128)[:2]))())
except Exception:
    _W_6464 = None


def _sc_sample(featT, maskf, cls_flat, y0n, x0n, y1n, x1n, wy, wx,
               *, B, M, H, W, C, mh, mw):
    K = _NUM_POINTS // M
    P = H * W
    mask_sz = mh * mw
    n_pairs = B * M
    HALF = P // 2
    ROWS = H // 2
    CPR = W // _L
    KH = K // 2
    NSUB = 16

    nrow0 = int(y1n[ROWS - 1]) + 1
    lo1 = int(y0n[ROWS])
    nrows = max(nrow0, mh - lo1)
    lo_by_half = (0, min(lo1, mh - nrows))

    mesh = plsc.VectorSubcoreMesh(core_axis_name="c", subcore_axis_name="s",
                                  num_cores=2, num_subcores=NSUB)

    @functools.partial(
        pl.kernel,
        out_type=(
            jax.ShapeDtypeStruct((B * _NUM_POINTS, C), jnp.float32),
            jax.ShapeDtypeStruct((3 * B * _NUM_POINTS,), jnp.int32),
        ),
        mesh=mesh,
        compiler_params=pltpu.CompilerParams(needs_layout_passes=False),
        scratch_types=[
            pltpu.VMEM((nrows, mw), jnp.float32),
            pltpu.VMEM((H,), jnp.int32),
            pltpu.VMEM((H,), jnp.int32),
            pltpu.VMEM((W,), jnp.int32),
            pltpu.VMEM((W,), jnp.int32),
            pltpu.VMEM((H,), jnp.float32),
            pltpu.VMEM((W,), jnp.float32),
            pltpu.VMEM((K,), jnp.int32),
            pltpu.VMEM((K,), jnp.int32),
            pltpu.VMEM((_L,), jnp.int32),
            pltpu.VMEM((K,), jnp.int32),
            pltpu.VMEM((K,), jnp.int32),
            pltpu.VMEM((K,), jnp.int32),
            pltpu.VMEM((_L,), jnp.int32),
            pltpu.VMEM((_L,), jnp.int32),
            pltpu.VMEM((KH,), jnp.int32),
            pltpu.VMEM((KH,), jnp.int32),
            pltpu.VMEM((KH, C), jnp.float32),
            pltpu.VMEM((1, C), jnp.float32),
            pltpu.VMEM((KH,), jnp.int32),
            pltpu.VMEM((KH,), jnp.int32),
            pltpu.VMEM((KH,), jnp.int32),
            pltpu.VMEM_SHARED((NSUB * K,), jnp.int32),
            pltpu.VMEM_SHARED((NSUB * K,), jnp.int32),
            pltpu.VMEM_SHARED((NSUB * _L,), jnp.int32),
            pltpu.SemaphoreType.DMA,
        ],
    )
    def body(featT_h, maskf_h, cls_h, y0_h, y1_h, x0_h, x1_h, wy_h, wx_h,
             points_h, indices_h, mask_v, y0_v, y1_v, x0_v, x1_v,
             wy_v, wx_v, tloc_v, floc_v, cnt_v, t0_v, t1_v, f0_v, c0_v,
             c1_v, oidx_v, gidx_v, rows_v, cls_v, hh_v, ww_v, bb_v,
             sh_t, sh_f, sh_c, sem):
        s = lax.axis_index("s")
        core = lax.axis_index("c")
        pair = core * (n_pairs // 2) + (s // 2)
        half = s % 2
        b = pair // M
        j = pair % M

        rowlo = jnp.where(half == 0, lo_by_half[0], lo_by_half[1])

        ins = [
            pltpu.async_copy(
                maskf_h.at[b, j, pl.ds(rowlo, nrows), :], mask_v, sem),
            pltpu.async_copy(y0_h, y0_v, sem),
            pltpu.async_copy(y1_h, y1_v, sem),
            pltpu.async_copy(x0_h, x0_v, sem),
            pltpu.async_copy(x1_h, x1_v, sem),
            pltpu.async_copy(wy_h, wy_v, sem),
            pltpu.async_copy(wx_h, wx_v, sem),
            pltpu.async_copy(cls_h.at[pl.ds(0, 1), :], cls_v, sem),
        ]
        for h in ins:
            h.wait()

        iota = lax.broadcasted_iota(jnp.int32, (_L,), 0)
        zero = jnp.zeros((_L,), jnp.int32)
        ones = zero + 1
        x0c = [x0_v[pl.ds(q * _L, _L)] for q in range(CPR)]
        x1c = [x1_v[pl.ds(q * _L, _L)] for q in range(CPR)]
        wxc = [wx_v[pl.ds(q * _L, _L)] for q in range(CPR)]
        axc = [1.0 - w for w in wxc]

        def row_body(r, ct):
            y = half * ROWS + r
            zy = zero + y
            y0s = plsc.load_gather(y0_v, [zy])
            y1s = plsc.load_gather(y1_v, [zy])
            wys = plsc.load_gather(wy_v, [zy])
            y0r = y0s - rowlo
            y1r = y1s - rowlo
            ay = 1.0 - wys
            for q in range(CPR):
                v00 = plsc.load_gather(mask_v, [y0r, x0c[q]])
                v01 = plsc.load_gather(mask_v, [y0r, x1c[q]])
                v10 = plsc.load_gather(mask_v, [y1r, x0c[q]])
                v11 = plsc.load_gather(mask_v, [y1r, x1c[q]])
                t1 = (v00 * ay) * axc[q]
                t2 = (v01 * ay) * wxc[q]
                t3 = (v10 * wys) * axc[q]
                t4 = (v11 * wys) * wxc[q]
                val = ((t1 + t2) + t3) + t4
                m = val > 0.5
                lpos = (r * W + q * _L) + iota
                rt = plsc.cumsum(ones, mask=m) + ct
                rf = (lpos + 1) - rt
                gpos = lpos + half * HALF
                plsc.store_scatter(tloc_v, [rt - 1], gpos,
                                   mask=m & (rt <= K))
                plsc.store_scatter(floc_v, [rf - 1], gpos,
                                   mask=(~m) & (rf <= K))
                ct = ct + plsc.all_reduce_population_count(m)
            return ct

        Tv = plsc.parallel_loop(
            0, ROWS, 1, unroll=2,
            carry=jnp.zeros((_L,), jnp.int32))(row_body)

        cnt_v[...] = Tv
        pltpu.sync_copy(tloc_v, sh_t.at[pl.ds(s * K, K)])
        pltpu.sync_copy(floc_v, sh_f.at[pl.ds(s * K, K)])
        pltpu.sync_copy(cnt_v, sh_c.at[pl.ds(s * _L, _L)])
        plsc.subcore_barrier()
        lo = (s // 2) * 2
        hi = lo + 1
        reads = [
            pltpu.async_copy(sh_t.at[pl.ds(lo * K, K)], t0_v, sem),
            pltpu.async_copy(sh_t.at[pl.ds(hi * K, K)], t1_v, sem),
            pltpu.async_copy(sh_f.at[pl.ds(lo * K, K)], f0_v, sem),
            pltpu.async_copy(sh_c.at[pl.ds(lo * _L, _L)], c0_v, sem),
            pltpu.async_copy(sh_c.at[pl.ds(hi * _L, _L)], c1_v, sem),
        ]
        for h in reads:
            h.wait()

        T0 = c0_v[...]
        T1 = c1_v[...]
        b1 = T0 + T1
        log2w = W.bit_length() - 1
        for q in range(KH // _L):
            sv = (half * KH + q * _L) + iota
            in0 = sv < T0
            in1 = (~in0) & (sv < b1)
            in2 = sv >= b1
            g0 = plsc.load_gather(t0_v, [jnp.clip(sv, 0, K - 1)], mask=in0)
            g1 = plsc.load_gather(t1_v, [jnp.clip(sv - T0, 0, K - 1)],
                                  mask=in1)
            g2 = plsc.load_gather(f0_v, [jnp.clip(sv - b1, 0, K - 1)],
                                  mask=in2)
            iv = jnp.where(in0, g0, jnp.where(in1, g1, g2))
            sl = pl.ds(q * _L, _L)
            oidx_v[sl] = iv
            gidx_v[sl] = iv + b * P
            hh_v[sl] = lax.shift_right_logical(iv, log2w)
            ww_v[sl] = iv & (W - 1)
            bb_v[sl] = zero + b

        pltpu.async_copy(featT_h.at[gidx_v], rows_v, sem).wait()

        cls_chunks = [cls_v[0, pl.ds(t * _L, _L)] for t in range(C // _L)]

        @plsc.parallel_loop(0, KH, 1, unroll=2)
        def _addrow(r):
            for t in range(C // _L):
                rows_v[r, pl.ds(t * _L, _L)] += cls_chunks[t]

        col = pair * K + half * KH
        npts = B * _NUM_POINTS
        outs = [
            pltpu.async_copy(rows_v, points_h.at[pl.ds(col, KH)], sem),
            pltpu.async_copy(bb_v, indices_h.at[pl.ds(col, KH)], sem),
            pltpu.async_copy(hh_v, indices_h.at[pl.ds(npts + col, KH)],
                             sem),
            pltpu.async_copy(ww_v,
                             indices_h.at[pl.ds(2 * npts + col, KH)], sem),
        ]
        for h in outs:
            h.wait()

    return body(featT, maskf, cls_flat, y0n, y1n, x0n, x1n, wy, wx)


def kernel(feat, ior_masks, cls_table):
    B, C, H, W = feat.shape
    M = ior_masks.shape[1]
    mh, mw = ior_masks.shape[2], ior_masks.shape[3]

    featT = feat.transpose(0, 2, 3, 1).reshape(B * H * W, C)

    y0n, x0n, y1n, x1n = _int_tables(H, W, mh, mw)

    if (H, W, mh, mw) == (64, 64, 128, 128) and _W_6464 is not None:
        wy, wx = _W_6464
    else:
        wy, wx = _jax_weights(H, W, mh, mw, y0n, x0n)()

    points_flat, indices_flat = _sc_sample(
        featT, ior_masks, cls_table, y0n, x0n, y1n, x1n, wy, wx,
        B=B, M=M, H=H, W=W, C=C, mh=mh, mw=mw)
    return (points_flat.reshape(B, _NUM_POINTS, C),
            indices_flat.reshape(3, B * _NUM_POINTS))


# speedup vs baseline: 1.0260x; 1.0017x over previous
import functools

import jax
import jax.numpy as jnp
import numpy as np
from jax import lax
from jax.experimental import pallas as pl
from jax.experimental.pallas import tpu as pltpu
from jax.experimental.pallas import tpu_sc as plsc

_NUM_POINTS = 512
_L = 16


def _np_linspace0(stop, num):
    div = num - 1
    step = np.arange(div, dtype=np.float32) / np.float32(div)
    body = np.float32(stop) * step
    return np.concatenate([body, np.array([stop], np.float32)])


def _int_tables(H, W, mh, mw):
    ys = _np_linspace0(float(mh - 1), H)
    xs = _np_linspace0(float(mw - 1), W)
    y0 = np.clip(np.floor(ys).astype(np.int32), 0, mh - 1)
    x0 = np.clip(np.floor(xs).astype(np.int32), 0, mw - 1)
    y1 = np.clip(y0 + 1, 0, mh - 1)
    x1 = np.clip(x0 + 1, 0, mw - 1)
    return y0, x0, y1, x1


def _jax_weights(H, W, mh, mw, y0n, x0n):
    def f():
        ys = jnp.linspace(0.0, float(mh - 1), H)
        xs = jnp.linspace(0.0, float(mw - 1), W)
        wy = ys - jnp.asarray(y0n).astype(jnp.float32)
        wx = xs - jnp.asarray(x0n).astype(jnp.float32)
        return wy, wx
    return f


try:
    _W_6464 = tuple(
        np.asarray(a)
        for a in jax.jit(_jax_weights(64, 64, 128, 128,
                                      *_int_tables(64, 64, 128, 128)[:2]))())
except Exception:
    _W_6464 = None


def _sc_sample(featT, maskf, cls_flat, y0n, x0n, y1n, x1n, wy, wx,
               *, B, M, H, W, C, mh, mw):
    K = _NUM_POINTS // M
    P = H * W
    mask_sz = mh * mw
    n_pairs = B * M
    HALF = P // 2
    ROWS = H // 2
    CPR = W // _L
    KH = K // 2
    NSUB = 16

    nrow0 = int(y1n[ROWS - 1]) + 1
    lo1 = int(y0n[ROWS])
    nrows = max(nrow0, mh - lo1)
    lo_by_half = (0, min(lo1, mh - nrows))

    mesh = plsc.VectorSubcoreMesh(core_axis_name="c", subcore_axis_name="s",
                                  num_cores=2, num_subcores=NSUB)

    @functools.partial(
        pl.kernel,
        out_type=(
            jax.ShapeDtypeStruct((B * _NUM_POINTS, C), jnp.float32),
            jax.ShapeDtypeStruct((3 * B * _NUM_POINTS,), jnp.int32),
        ),
        mesh=mesh,
        compiler_params=pltpu.CompilerParams(needs_layout_passes=False),
        scratch_types=[
            pltpu.VMEM((nrows, mw), jnp.float32),
            pltpu.VMEM((H,), jnp.int32),
            pltpu.VMEM((H,), jnp.int32),
            pltpu.VMEM((W,), jnp.int32),
            pltpu.VMEM((W,), jnp.int32),
            pltpu.VMEM((H,), jnp.float32),
            pltpu.VMEM((W,), jnp.float32),
            pltpu.VMEM((K,), jnp.int32),
            pltpu.VMEM((K,), jnp.int32),
            pltpu.VMEM((_L,), jnp.int32),
            pltpu.VMEM((K,), jnp.int32),
            pltpu.VMEM((K,), jnp.int32),
            pltpu.VMEM((K,), jnp.int32),
            pltpu.VMEM((_L,), jnp.int32),
            pltpu.VMEM((_L,), jnp.int32),
            pltpu.VMEM((KH,), jnp.int32),
            pltpu.VMEM((KH, C), jnp.float32),
            pltpu.VMEM((1, C), jnp.float32),
            pltpu.VMEM((KH,), jnp.int32),
            pltpu.VMEM((KH,), jnp.int32),
            pltpu.VMEM((KH,), jnp.int32),
            pltpu.VMEM_SHARED((NSUB * K,), jnp.int32),
            pltpu.VMEM_SHARED((NSUB * K,), jnp.int32),
            pltpu.VMEM_SHARED((NSUB * _L,), jnp.int32),
            pltpu.SemaphoreType.DMA,
        ],
    )
    def body(featT_h, maskf_h, cls_h, y0_h, y1_h, x0_h, x1_h, wy_h, wx_h,
             points_h, indices_h, mask_v, y0_v, y1_v, x0_v, x1_v,
             wy_v, wx_v, tloc_v, floc_v, cnt_v, t0_v, t1_v, f0_v, c0_v,
             c1_v, gidx_v, rows_v, cls_v, hh_v, ww_v, bb_v,
             sh_t, sh_f, sh_c, sem):
        s = lax.axis_index("s")
        core = lax.axis_index("c")
        pair = core * (n_pairs // 2) + (s // 2)
        half = s % 2
        b = pair // M
        j = pair % M

        rowlo = jnp.where(half == 0, lo_by_half[0], lo_by_half[1])

        ins = [
            pltpu.async_copy(
                maskf_h.at[b, j, pl.ds(rowlo, nrows), :], mask_v, sem),
            pltpu.async_copy(y0_h, y0_v, sem),
            pltpu.async_copy(y1_h, y1_v, sem),
            pltpu.async_copy(x0_h, x0_v, sem),
            pltpu.async_copy(x1_h, x1_v, sem),
            pltpu.async_copy(wy_h, wy_v, sem),
            pltpu.async_copy(wx_h, wx_v, sem),
            pltpu.async_copy(cls_h.at[pl.ds(0, 1), :], cls_v, sem),
        ]
        for h in ins:
            h.wait()

        iota = lax.broadcasted_iota(jnp.int32, (_L,), 0)
        zero = jnp.zeros((_L,), jnp.int32)
        ones = zero + 1
        x0c = [x0_v[pl.ds(q * _L, _L)] for q in range(CPR)]
        x1c = [x1_v[pl.ds(q * _L, _L)] for q in range(CPR)]
        wxc = [wx_v[pl.ds(q * _L, _L)] for q in range(CPR)]
        axc = [1.0 - w for w in wxc]

        def row_body(r, ct):
            y = half * ROWS + r
            zy = zero + y
            y0s = plsc.load_gather(y0_v, [zy])
            y1s = plsc.load_gather(y1_v, [zy])
            wys = plsc.load_gather(wy_v, [zy])
            y0r = y0s - rowlo
            y1r = y1s - rowlo
            ay = 1.0 - wys
            for q in range(CPR):
                v00 = plsc.load_gather(mask_v, [y0r, x0c[q]])
                v01 = plsc.load_gather(mask_v, [y0r, x1c[q]])
                v10 = plsc.load_gather(mask_v, [y1r, x0c[q]])
                v11 = plsc.load_gather(mask_v, [y1r, x1c[q]])
                t1 = (v00 * ay) * axc[q]
                t2 = (v01 * ay) * wxc[q]
                t3 = (v10 * wys) * axc[q]
                t4 = (v11 * wys) * wxc[q]
                val = ((t1 + t2) + t3) + t4
                m = val > 0.5
                lpos = (r * W + q * _L) + iota
                rt = plsc.cumsum(ones, mask=m) + ct
                rf = (lpos + 1) - rt
                gpos = lpos + half * HALF
                plsc.store_scatter(tloc_v, [rt - 1], gpos,
                                   mask=m & (rt <= K))
                plsc.store_scatter(floc_v, [rf - 1], gpos,
                                   mask=(~m) & (rf <= K))
                ct = ct + plsc.all_reduce_population_count(m)
            return ct

        Tv = plsc.parallel_loop(
            0, ROWS, 1, unroll=2,
            carry=jnp.zeros((_L,), jnp.int32))(row_body)

        cnt_v[...] = Tv
        pltpu.sync_copy(tloc_v, sh_t.at[pl.ds(s * K, K)])
        pltpu.sync_copy(floc_v, sh_f.at[pl.ds(s * K, K)])
        pltpu.sync_copy(cnt_v, sh_c.at[pl.ds(s * _L, _L)])
        plsc.subcore_barrier()
        lo = (s // 2) * 2
        hi = lo + 1
        reads = [
            pltpu.async_copy(sh_t.at[pl.ds(lo * K, K)], t0_v, sem),
            pltpu.async_copy(sh_t.at[pl.ds(hi * K, K)], t1_v, sem),
            pltpu.async_copy(sh_f.at[pl.ds(lo * K, K)], f0_v, sem),
            pltpu.async_copy(sh_c.at[pl.ds(lo * _L, _L)], c0_v, sem),
            pltpu.async_copy(sh_c.at[pl.ds(hi * _L, _L)], c1_v, sem),
        ]
        for h in reads:
            h.wait()

        T0 = c0_v[...]
        T1 = c1_v[...]
        b1 = T0 + T1
        log2w = W.bit_length() - 1
        for q in range(KH // _L):
            sv = (half * KH + q * _L) + iota
            in0 = sv < T0
            in1 = (~in0) & (sv < b1)
            in2 = sv >= b1
            g0 = plsc.load_gather(t0_v, [jnp.clip(sv, 0, K - 1)], mask=in0)
            g1 = plsc.load_gather(t1_v, [jnp.clip(sv - T0, 0, K - 1)],
                                  mask=in1)
            g2 = plsc.load_gather(f0_v, [jnp.clip(sv - b1, 0, K - 1)],
                                  mask=in2)
            iv = jnp.where(in0, g0, jnp.where(in1, g1, g2))
            sl = pl.ds(q * _L, _L)
            gidx_v[sl] = iv + b * P
            hh_v[sl] = lax.shift_right_logical(iv, log2w)
            ww_v[sl] = iv & (W - 1)
            bb_v[sl] = zero + b

        pltpu.async_copy(featT_h.at[gidx_v], rows_v, sem).wait()

        cls_chunks = [cls_v[0, pl.ds(t * _L, _L)] for t in range(C // _L)]

        @plsc.parallel_loop(0, KH, 1, unroll=2)
        def _addrow(r):
            for t in range(C // _L):
                rows_v[r, pl.ds(t * _L, _L)] += cls_chunks[t]

        col = pair * K + half * KH
        npts = B * _NUM_POINTS
        outs = [
            pltpu.async_copy(rows_v, points_h.at[pl.ds(col, KH)], sem),
            pltpu.async_copy(bb_v, indices_h.at[pl.ds(col, KH)], sem),
            pltpu.async_copy(hh_v, indices_h.at[pl.ds(npts + col, KH)],
                             sem),
            pltpu.async_copy(ww_v,
                             indices_h.at[pl.ds(2 * npts + col, KH)], sem),
        ]
        for h in outs:
            h.wait()

    return body(featT, maskf, cls_flat, y0n, y1n, x0n, x1n, wy, wx)


def kernel(feat, ior_masks, cls_table):
    B, C, H, W = feat.shape
    M = ior_masks.shape[1]
    mh, mw = ior_masks.shape[2], ior_masks.shape[3]

    featT = feat.transpose(0, 2, 3, 1).reshape(B * H * W, C)

    y0n, x0n, y1n, x1n = _int_tables(H, W, mh, mw)

    if (H, W, mh, mw) == (64, 64, 128, 128) and _W_6464 is not None:
        wy, wx = _W_6464
    else:
        wy, wx = _jax_weights(H, W, mh, mw, y0n, x0n)()

    points_flat, indices_flat = _sc_sample(
        featT, ior_masks, cls_table, y0n, x0n, y1n, x1n, wy, wx,
        B=B, M=M, H=H, W=W, C=C, mh=mh, mw=mw)
    return (points_flat.reshape(B, _NUM_POINTS, C),
            indices_flat.reshape(3, B * _NUM_POINTS))
